# baseline (speedup 1.0000x reference)
import jax
import jax.numpy as jnp
from jax.experimental import pallas as pl
from jax.experimental.pallas import tpu as pltpu

_BM = 512
_XSLOTS = 3


def _linear_kernel(x_hbm, w_ref, b_ref, o_hbm, x_buf, o_buf, x_sems, o_sems):
    m = x_hbm.shape[0]
    n_tiles = m // _BM

    def x_copy(i):
        return pltpu.make_async_copy(
            x_hbm.at[pl.ds(i * _BM, _BM), :],
            x_buf.at[i % _XSLOTS],
            x_sems.at[i % _XSLOTS],
        )

    def o_copy(i):
        return pltpu.make_async_copy(
            o_buf.at[i % 2],
            o_hbm.at[pl.ds(i * _BM, _BM), :],
            o_sems.at[i % 2],
        )

    for i in range(min(_XSLOTS, n_tiles)):
        x_copy(i).start()

    for i in range(n_tiles):
        x_copy(i).wait()
        acc = jax.lax.dot_general(
            x_buf[i % _XSLOTS], w_ref[...],
            dimension_numbers=(((1,), (1,)), ((), ())),
            preferred_element_type=jnp.float32,
        )
        if i >= 2:
            o_copy(i - 2).wait()
        o_buf[i % 2] = acc + b_ref[...]
        o_copy(i).start()
        if i + _XSLOTS < n_tiles:
            x_copy(i + _XSLOTS).start()

    for i in range(max(0, n_tiles - 2), n_tiles):
        o_copy(i).wait()


def kernel(x, weights, bias):
    m, k = x.shape
    n = weights.shape[0]
    bias2d = bias.reshape(1, n)
    return pl.pallas_call(
        _linear_kernel,
        in_specs=[
            pl.BlockSpec(memory_space=pl.ANY),
            pl.BlockSpec(memory_space=pltpu.VMEM),
            pl.BlockSpec(memory_space=pltpu.VMEM),
        ],
        out_specs=pl.BlockSpec(memory_space=pl.ANY),
        out_shape=jax.ShapeDtypeStruct((m, n), x.dtype),
        scratch_shapes=[
            pltpu.VMEM((_XSLOTS, _BM, k), jnp.float32),
            pltpu.VMEM((2, _BM, n), jnp.float32),
            pltpu.SemaphoreType.DMA((_XSLOTS,)),
            pltpu.SemaphoreType.DMA((2,)),
        ],
        compiler_params=pltpu.CompilerParams(
            vmem_limit_bytes=56 * 1024 * 1024,
        ),
        name="linear_bias",
    )(x, weights, bias2d)


# speedup vs baseline: 1.0884x; 1.0884x over previous
import jax
import jax.numpy as jnp
from jax.experimental import pallas as pl
from jax.experimental.pallas import tpu as pltpu

_BM = 512
_XSLOTS = 3


def _linear_kernel(x_hbm, w_ref, b_ref, o_hbm, x_buf, o_buf, x_sems, o_sems):
    m = x_hbm.shape[0]
    n_tiles = m // _BM

    def x_copy(i):
        return pltpu.make_async_copy(
            x_hbm.at[pl.ds(i * _BM, _BM), :],
            x_buf.at[i % _XSLOTS],
            x_sems.at[i % _XSLOTS],
        )

    def o_copy(i):
        return pltpu.make_async_copy(
            o_buf.at[i % 2],
            o_hbm.at[pl.ds(i * _BM, _BM), :],
            o_sems.at[i % 2],
        )

    for i in range(min(_XSLOTS, n_tiles)):
        x_copy(i).start()

    def body(i, carry):
        x_copy(i).wait()
        acc = jax.lax.dot_general(
            x_buf[i % _XSLOTS], w_ref[...],
            dimension_numbers=(((1,), (1,)), ((), ())),
            preferred_element_type=jnp.float32,
        )

        @pl.when(i >= 2)
        def _():
            o_copy(i - 2).wait()

        o_buf[i % 2] = acc + b_ref[...]
        o_copy(i).start()

        @pl.when(i + _XSLOTS < n_tiles)
        def _():
            x_copy(i + _XSLOTS).start()

        return carry

    jax.lax.fori_loop(0, n_tiles, body, 0)

    for i in range(max(0, n_tiles - 2), n_tiles):
        o_copy(i).wait()


def kernel(x, weights, bias):
    m, k = x.shape
    n = weights.shape[0]
    bias2d = bias.reshape(1, n)
    return pl.pallas_call(
        _linear_kernel,
        in_specs=[
            pl.BlockSpec(memory_space=pl.ANY),
            pl.BlockSpec(memory_space=pltpu.VMEM),
            pl.BlockSpec(memory_space=pltpu.VMEM),
        ],
        out_specs=pl.BlockSpec(memory_space=pl.ANY),
        out_shape=jax.ShapeDtypeStruct((m, n), x.dtype),
        scratch_shapes=[
            pltpu.VMEM((_XSLOTS, _BM, k), jnp.float32),
            pltpu.VMEM((2, _BM, n), jnp.float32),
            pltpu.SemaphoreType.DMA((_XSLOTS,)),
            pltpu.SemaphoreType.DMA((2,)),
        ],
        compiler_params=pltpu.CompilerParams(
            vmem_limit_bytes=56 * 1024 * 1024,
        ),
        name="linear_bias",
    )(x, weights, bias2d)


# speedup vs baseline: 1.2790x; 1.1751x over previous
import jax
import jax.numpy as jnp
from jax.experimental import pallas as pl
from jax.experimental.pallas import tpu as pltpu

_BM = 512
_XSLOTS = 3


def _linear_kernel(x_hbm, w_ref, b_ref, o_hbm, x_buf, o_buf, x_sems, o_sems):
    m = x_hbm.shape[0]
    n_tiles = m // _BM

    def x_copy(i):
        return pltpu.make_async_copy(
            x_hbm.at[pl.ds(i * _BM, _BM), :],
            x_buf.at[i % _XSLOTS],
            x_sems.at[i % _XSLOTS],
        )

    def o_copy(i):
        return pltpu.make_async_copy(
            o_buf.at[i % 2],
            o_hbm.at[pl.ds(i * _BM, _BM), :],
            o_sems.at[i % 2],
        )

    for i in range(min(_XSLOTS, n_tiles)):
        x_copy(i).start()

    def body(i, carry):
        x_copy(i).wait()
        acc = jax.lax.dot_general(
            x_buf[i % _XSLOTS], w_ref[...],
            dimension_numbers=(((1,), (1,)), ((), ())),
            preferred_element_type=jnp.float32,
        )

        @pl.when(i >= 2)
        def _():
            o_copy(i - 2).wait()

        for c in range(0, acc.shape[1], 512):
            sl = pl.ds(c, 512)
            o_buf[i % 2, :, sl] = acc[:, c:c + 512] + b_ref[:, sl]
        o_copy(i).start()

        @pl.when(i + _XSLOTS < n_tiles)
        def _():
            x_copy(i + _XSLOTS).start()

        return carry

    jax.lax.fori_loop(0, n_tiles, body, 0)

    for i in range(max(0, n_tiles - 2), n_tiles):
        o_copy(i).wait()


def kernel(x, weights, bias):
    m, k = x.shape
    n = weights.shape[0]
    bias2d = bias.reshape(1, n)
    return pl.pallas_call(
        _linear_kernel,
        in_specs=[
            pl.BlockSpec(memory_space=pl.ANY),
            pl.BlockSpec(memory_space=pltpu.VMEM),
            pl.BlockSpec(memory_space=pltpu.VMEM),
        ],
        out_specs=pl.BlockSpec(memory_space=pl.ANY),
        out_shape=jax.ShapeDtypeStruct((m, n), x.dtype),
        scratch_shapes=[
            pltpu.VMEM((_XSLOTS, _BM, k), jnp.float32),
            pltpu.VMEM((2, _BM, n), jnp.float32),
            pltpu.SemaphoreType.DMA((_XSLOTS,)),
            pltpu.SemaphoreType.DMA((2,)),
        ],
        compiler_params=pltpu.CompilerParams(
            vmem_limit_bytes=56 * 1024 * 1024,
        ),
        name="linear_bias",
    )(x, weights, bias2d)


# speedup vs baseline: 1.3148x; 1.0280x over previous
import jax
import jax.numpy as jnp
from jax.experimental import pallas as pl
from jax.experimental.pallas import tpu as pltpu

_BM = 512
_XSLOTS = 3
_WCHUNK = 512


def _dot(xb, wb):
    return jax.lax.dot_general(
        xb, wb,
        dimension_numbers=(((1,), (1,)), ((), ())),
        preferred_element_type=jnp.float32,
    )


def _linear_kernel(x_hbm, w_hbm, b_ref, o_hbm,
                   x_buf, w_vmem, o_buf, x_sems, w_sems, o_sems):
    m = x_hbm.shape[0]
    n = w_hbm.shape[0]
    n_tiles = m // _BM
    n_wchunks = n // _WCHUNK

    def x_copy(i):
        return pltpu.make_async_copy(
            x_hbm.at[pl.ds(i * _BM, _BM), :],
            x_buf.at[i % _XSLOTS],
            x_sems.at[i % _XSLOTS],
        )

    def w_copy(c):
        return pltpu.make_async_copy(
            w_hbm.at[pl.ds(c * _WCHUNK, _WCHUNK), :],
            w_vmem.at[pl.ds(c * _WCHUNK, _WCHUNK), :],
            w_sems.at[c],
        )

    def o_copy(i):
        return pltpu.make_async_copy(
            o_buf.at[i % 2],
            o_hbm.at[pl.ds(i * _BM, _BM), :],
            o_sems.at[i % 2],
        )

    for i in range(min(_XSLOTS, n_tiles)):
        x_copy(i).start()
    for c in range(n_wchunks):
        w_copy(c).start()

    x_copy(0).wait()
    for c in range(n_wchunks):
        w_copy(c).wait()
        sl = pl.ds(c * _WCHUNK, _WCHUNK)
        o_buf[0, :, sl] = _dot(x_buf[0], w_vmem[sl, :]) + b_ref[:, sl]
    o_copy(0).start()
    if _XSLOTS < n_tiles:
        x_copy(_XSLOTS).start()

    def body(i, carry):
        x_copy(i).wait()
        acc = _dot(x_buf[i % _XSLOTS], w_vmem[...])

        @pl.when(i >= 2)
        def _():
            o_copy(i - 2).wait()

        for c in range(0, n, _WCHUNK):
            sl = pl.ds(c, _WCHUNK)
            o_buf[i % 2, :, sl] = acc[:, c:c + _WCHUNK] + b_ref[:, sl]
        o_copy(i).start()

        @pl.when(i + _XSLOTS < n_tiles)
        def _():
            x_copy(i + _XSLOTS).start()

        return carry

    jax.lax.fori_loop(1, n_tiles, body, 0)

    for i in range(max(0, n_tiles - 2), n_tiles):
        o_copy(i).wait()


def kernel(x, weights, bias):
    m, k = x.shape
    n = weights.shape[0]
    bias2d = bias.reshape(1, n)
    return pl.pallas_call(
        _linear_kernel,
        in_specs=[
            pl.BlockSpec(memory_space=pl.ANY),
            pl.BlockSpec(memory_space=pl.ANY),
            pl.BlockSpec(memory_space=pltpu.VMEM),
        ],
        out_specs=pl.BlockSpec(memory_space=pl.ANY),
        out_shape=jax.ShapeDtypeStruct((m, n), x.dtype),
        scratch_shapes=[
            pltpu.VMEM((_XSLOTS, _BM, k), jnp.float32),
            pltpu.VMEM((n, k), jnp.float32),
            pltpu.VMEM((2, _BM, n), jnp.float32),
            pltpu.SemaphoreType.DMA((_XSLOTS,)),
            pltpu.SemaphoreType.DMA((n // _WCHUNK,)),
            pltpu.SemaphoreType.DMA((2,)),
        ],
        compiler_params=pltpu.CompilerParams(
            vmem_limit_bytes=56 * 1024 * 1024,
        ),
        name="linear_bias",
    )(x, weights, bias2d)


# speedup vs baseline: 1.3488x; 1.0259x over previous
import jax
import jax.numpy as jnp
from jax.experimental import pallas as pl
from jax.experimental.pallas import tpu as pltpu

_BM = 512


def _linear_kernel(x_ref, w_ref, b_ref, o_ref):
    acc = jax.lax.dot_general(
        x_ref[...], w_ref[...],
        dimension_numbers=(((1,), (1,)), ((), ())),
        preferred_element_type=jnp.float32,
    )
    o_ref[...] = acc + b_ref[...]


def kernel(x, weights, bias):
    m, k = x.shape
    n = weights.shape[0]
    bias2d = bias.reshape(1, n)
    grid = (m // _BM,)
    return pl.pallas_call(
        _linear_kernel,
        grid=grid,
        in_specs=[
            pl.BlockSpec((_BM, k), lambda i: (i, 0)),
            pl.BlockSpec((n, k), lambda i: (0, 0)),
            pl.BlockSpec((1, n), lambda i: (0, 0)),
        ],
        out_specs=pl.BlockSpec((_BM, n), lambda i: (i, 0)),
        out_shape=jax.ShapeDtypeStruct((m, n), x.dtype),
        compiler_params=pltpu.CompilerParams(
            dimension_semantics=("parallel",),
            vmem_limit_bytes=56 * 1024 * 1024,
        ),
        name="linear_bias",
    )(x, weights, bias2d)
